# baseline (speedup 1.0000x reference)
import functools

import jax
import jax.numpy as jnp
from jax import lax
from jax.experimental import pallas as pl
from jax.experimental.pallas import tpu as pltpu
from jax.experimental.pallas import tpu_sc as plsc

_TN = 512
_NC = 2
_NS = 16
_LANES = 16

_DN_NT = (((1,), (1,)), ((), ()))
_DN_TN = (((0,), (0,)), ((), ()))


def _gelu_exact(v):
    return 0.5 * v * (1.0 + jax.lax.erf(v * jnp.float32(0.7071067811865476)))


def _logits_body(x_ref, wd_ref, rmat_ref, logits_ref):
    rmat2 = lax.dot_general(wd_ref[...], rmat_ref[...], _DN_TN,
                            preferred_element_type=jnp.float32)
    logits_ref[...] = lax.dot_general(
        rmat2, x_ref[...], (((0,), (1,)), ((), ())),
        preferred_element_type=jnp.float32)


def _router_body(n_chunks, n_tokens, E, lg_hbm, gate_hbm, pp_hbm, lp_hbm,
                 lg_v, gt_v, pp_v, lp_v):
    wid = lax.axis_index("s") * _NC + lax.axis_index("c")
    tpw = n_chunks * _LANES
    base = wid * tpw
    for e in range(E):
        pltpu.sync_copy(lg_hbm.at[pl.ds(e * n_tokens + base, tpw)],
                        lg_v.at[pl.ds(e * tpw, tpw)])

    zf = jnp.zeros((_LANES,), jnp.float32)
    acc_p = [zf] * E
    acc_l = [zf] * E
    for j in range(n_chunks):
        off = j * _LANES
        ps = [lg_v[pl.ds(e * tpw + off, _LANES)] for e in range(E)]
        mx = ps[0]
        for e in range(1, E):
            mx = jnp.maximum(mx, ps[e])
        ps = [jnp.exp(v - mx) for v in ps]
        tot = ps[0]
        for e in range(1, E):
            tot = tot + ps[e]
        ps = [v / tot for v in ps]
        m1 = ps[0]
        for e in range(1, E):
            m1 = jnp.maximum(m1, ps[e])
        i1 = jnp.zeros((_LANES,), jnp.int32)
        for e in range(E - 1, -1, -1):
            i1 = jnp.where(ps[e] == m1, e, i1)
        rest = [jnp.where(i1 == e, -1.0, ps[e]) for e in range(E)]
        m2 = rest[0]
        for e in range(1, E):
            m2 = jnp.maximum(m2, rest[e])
        i2 = jnp.zeros((_LANES,), jnp.int32)
        for e in range(E - 1, -1, -1):
            i2 = jnp.where(rest[e] == m2, e, i2)
        denom = m1 + m2
        for e in range(E):
            sel = (i1 == e) | (i2 == e)
            gt_v[pl.ds(e * tpw + off, _LANES)] = (
                jnp.where(sel, ps[e], 0.0) / denom)
            acc_p[e] = acc_p[e] + ps[e]
            acc_l[e] = acc_l[e] + jnp.where(sel, 1.0, 0.0)

    for e in range(E):
        pp_v[pl.ds(e * _LANES, _LANES)] = acc_p[e]
        lp_v[pl.ds(e * _LANES, _LANES)] = acc_l[e]
    for e in range(E):
        pltpu.sync_copy(gt_v.at[pl.ds(e * tpw, tpw)],
                        gate_hbm.at[pl.ds(e * n_tokens + base, tpw)])
    npad = E * _LANES
    pltpu.sync_copy(pp_v, pp_hbm.at[pl.ds(wid * npad, npad)])
    pltpu.sync_copy(lp_v, lp_hbm.at[pl.ds(wid * npad, npad)])


def _mix_body(nsteps, n_tokens, x_ref, w1_ref, b1_ref, w2_ref, b2_ref,
              wd_ref, sel_ref, wup_ref, gate_ref, pp_ref, lp_ref,
              out_ref, aux_ref):
    step = pl.program_id(0)
    xb = x_ref[...]
    E = sel_ref.shape[0]

    xb_h = xb.astype(jnp.bfloat16)
    h = _gelu_exact(
        lax.dot_general(xb_h, w1_ref[...], _DN_NT,
                        preferred_element_type=jnp.float32)
        + b1_ref[...])
    shared = (lax.dot_general(h.astype(jnp.bfloat16), w2_ref[...], _DN_NT,
                              preferred_element_type=jnp.float32)
              + b2_ref[...])

    feats = lax.dot_general(xb_h, wd_ref[...], _DN_NT,
                            preferred_element_type=jnp.float32)
    gate_big = lax.dot_general(gate_ref[...], sel_ref[...], _DN_TN,
                               preferred_element_type=jnp.float32)
    wf = _gelu_exact(feats) * gate_big
    aoe = jnp.dot(wf.astype(jnp.bfloat16), wup_ref[...],
                  preferred_element_type=jnp.float32)

    out_ref[...] = xb + shared + aoe

    @pl.when(step == nsteps - 1)
    def _fin():
        lid = jax.lax.broadcasted_iota(jnp.int32, (E * _LANES, E), 0)
        eid = jax.lax.broadcasted_iota(jnp.int32, (E * _LANES, E), 1)
        selk = (lid // _LANES == eid).astype(jnp.float32)
        psum = jnp.dot(jnp.sum(pp_ref[...], axis=0, keepdims=True), selk,
                       preferred_element_type=jnp.float32)
        lsum = jnp.dot(jnp.sum(lp_ref[...], axis=0, keepdims=True), selk,
                       preferred_element_type=jnp.float32)
        n_f = jnp.float32(n_tokens)
        aux_ref[...] = (jnp.float32(E) / (n_f * n_f)
                        * jnp.sum(psum * lsum, keepdims=True))


def kernel(x, conv1_w, conv1_b, conv2_w, conv2_b, w_down, router_w, w_up):
    B, C, H, W = x.shape
    E, d_low, _ = w_up.shape
    hid = conv1_w.shape[0]
    N = B * H * W
    nsteps = N // _TN
    nw = _NC * _NS
    tpw = N // nw
    n_chunks = tpw // _LANES
    assert N % _TN == 0 and N % (nw * _LANES) == 0

    x_tok = x.transpose(0, 2, 3, 1).reshape(N, C)
    w1b = conv1_w.astype(jnp.bfloat16)
    w2b = conv2_w.astype(jnp.bfloat16)
    wdb = w_down.astype(jnp.bfloat16)
    eye = jnp.eye(E, dtype=x.dtype)
    rmat = jnp.kron(eye, router_w[0][:, None])
    selm = jnp.kron(eye, jnp.ones((1, d_low), x.dtype))
    wupf = w_up.reshape(E * d_low, C).astype(jnp.bfloat16)

    full = lambda r, c: pl.BlockSpec((r, c), lambda i: (0, 0))

    tn_lg = 1024
    logits = pl.pallas_call(
        _logits_body,
        grid=(N // tn_lg,),
        in_specs=[
            pl.BlockSpec((tn_lg, C), lambda i: (i, 0)),
            full(E * d_low, C), full(E * d_low, E),
        ],
        out_specs=pl.BlockSpec((E, tn_lg), lambda i: (0, i)),
        out_shape=jax.ShapeDtypeStruct((E, N), jnp.float32),
    )(x_tok, w_down, rmat)

    mesh = plsc.VectorSubcoreMesh(core_axis_name="c", subcore_axis_name="s")
    gate_flat, pp, lp = pl.kernel(
        functools.partial(_router_body, n_chunks, N, E),
        out_type=[
            jax.ShapeDtypeStruct((N * E,), jnp.float32),
            jax.ShapeDtypeStruct((nw * E * _LANES,), jnp.float32),
            jax.ShapeDtypeStruct((nw * E * _LANES,), jnp.float32),
        ],
        mesh=mesh,
        scratch_types=[
            pltpu.VMEM((tpw * E,), jnp.float32),
            pltpu.VMEM((tpw * E,), jnp.float32),
            pltpu.VMEM((E * _LANES,), jnp.float32),
            pltpu.VMEM((E * _LANES,), jnp.float32),
        ],
    )(logits.reshape(N * E))

    gate = gate_flat.reshape(E, N)
    pp2 = pp.reshape(nw, E * _LANES)
    lp2 = lp.reshape(nw, E * _LANES)

    tn_mix = 2304
    nsteps_mix = N // tn_mix
    out_tok, aux = pl.pallas_call(
        functools.partial(_mix_body, nsteps_mix, N),
        grid=(nsteps_mix,),
        in_specs=[
            pl.BlockSpec((tn_mix, C), lambda i: (i, 0)),
            full(hid, C), full(1, hid), full(C, hid), full(1, C),
            full(E * d_low, C), full(E, E * d_low), full(E * d_low, C),
            pl.BlockSpec((E, tn_mix), lambda i: (0, i)),
            full(nw, E * _LANES), full(nw, E * _LANES),
        ],
        out_specs=[
            pl.BlockSpec((tn_mix, C), lambda i: (i, 0)),
            full(1, 1),
        ],
        out_shape=[
            jax.ShapeDtypeStruct((N, C), jnp.float32),
            jax.ShapeDtypeStruct((1, 1), jnp.float32),
        ],
    )(x_tok, w1b, conv1_b[None, :], w2b, conv2_b[None, :], wdb, selm, wupf,
      gate, pp2, lp2)

    out = out_tok.reshape(B, H, W, C).transpose(0, 3, 1, 2)
    return (out, aux[0, 0])


# speedup vs baseline: 1.0139x; 1.0139x over previous
import functools

import jax
import jax.numpy as jnp
from jax import lax
from jax.experimental import pallas as pl
from jax.experimental.pallas import tpu as pltpu
from jax.experimental.pallas import tpu_sc as plsc

_TN = 512
_NC = 2
_NS = 16
_LANES = 16

_DN_NT = (((1,), (1,)), ((), ()))
_DN_TN = (((0,), (0,)), ((), ()))


def _gelu_exact(v):
    return 0.5 * v * (1.0 + jax.lax.erf(v * jnp.float32(0.7071067811865476)))


def _logits_body(x_ref, wd_ref, rmat_ref, logits_ref):
    rmat2 = lax.dot_general(wd_ref[...], rmat_ref[...], _DN_TN,
                            preferred_element_type=jnp.float32)
    logits_ref[...] = lax.dot_general(
        rmat2, x_ref[...], (((0,), (1,)), ((), ())),
        preferred_element_type=jnp.float32)


def _router_body(n_chunks, n_tokens, E, lg_hbm, gate_hbm, pp_hbm, lp_hbm,
                 lg_v, gt_v, pp_v, lp_v):
    wid = lax.axis_index("s") * _NC + lax.axis_index("c")
    tpw = n_chunks * _LANES
    base = wid * tpw
    for e in range(E):
        pltpu.sync_copy(lg_hbm.at[pl.ds(e * n_tokens + base, tpw)],
                        lg_v.at[pl.ds(e * tpw, tpw)])

    zf = jnp.zeros((_LANES,), jnp.float32)
    acc_p = [zf] * E
    acc_l = [zf] * E
    for j in range(n_chunks):
        off = j * _LANES
        ps = [lg_v[pl.ds(e * tpw + off, _LANES)] for e in range(E)]
        mx = ps[0]
        for e in range(1, E):
            mx = jnp.maximum(mx, ps[e])
        ps = [jnp.exp(v - mx) for v in ps]
        tot = ps[0]
        for e in range(1, E):
            tot = tot + ps[e]
        ps = [v / tot for v in ps]
        m1 = ps[0]
        for e in range(1, E):
            m1 = jnp.maximum(m1, ps[e])
        i1 = jnp.zeros((_LANES,), jnp.int32)
        for e in range(E - 1, -1, -1):
            i1 = jnp.where(ps[e] == m1, e, i1)
        rest = [jnp.where(i1 == e, -1.0, ps[e]) for e in range(E)]
        m2 = rest[0]
        for e in range(1, E):
            m2 = jnp.maximum(m2, rest[e])
        i2 = jnp.zeros((_LANES,), jnp.int32)
        for e in range(E - 1, -1, -1):
            i2 = jnp.where(rest[e] == m2, e, i2)
        denom = m1 + m2
        for e in range(E):
            sel = (i1 == e) | (i2 == e)
            gt_v[pl.ds(e * tpw + off, _LANES)] = (
                jnp.where(sel, ps[e], 0.0) / denom)
            acc_p[e] = acc_p[e] + ps[e]
            acc_l[e] = acc_l[e] + jnp.where(sel, 1.0, 0.0)

    for e in range(E):
        pp_v[pl.ds(e * _LANES, _LANES)] = acc_p[e]
        lp_v[pl.ds(e * _LANES, _LANES)] = acc_l[e]
    for e in range(E):
        pltpu.sync_copy(gt_v.at[pl.ds(e * tpw, tpw)],
                        gate_hbm.at[pl.ds(e * n_tokens + base, tpw)])
    npad = E * _LANES
    pltpu.sync_copy(pp_v, pp_hbm.at[pl.ds(wid * npad, npad)])
    pltpu.sync_copy(lp_v, lp_hbm.at[pl.ds(wid * npad, npad)])


def _mix_body(nsteps, n_tokens, x_ref, w1_ref, b1_ref, w2_ref, b2_ref,
              wd_ref, sel_ref, wup_ref, gate_ref, pp_ref, lp_ref,
              out_ref, aux_ref):
    step = pl.program_id(0)
    xb = x_ref[...]
    E = sel_ref.shape[0]

    xb_h = xb.astype(jnp.bfloat16)
    h = _gelu_exact(
        lax.dot_general(xb_h, w1_ref[...], _DN_NT,
                        preferred_element_type=jnp.float32)
        + b1_ref[...])
    shared = (lax.dot_general(h.astype(jnp.bfloat16), w2_ref[...], _DN_NT,
                              preferred_element_type=jnp.float32)
              + b2_ref[...])

    feats = lax.dot_general(xb_h, wd_ref[...], _DN_NT,
                            preferred_element_type=jnp.float32)
    gate_big = lax.dot_general(gate_ref[...], sel_ref[...], _DN_TN,
                               preferred_element_type=jnp.float32)
    wf = _gelu_exact(feats) * gate_big
    aoe = jnp.dot(wf.astype(jnp.bfloat16), wup_ref[...],
                  preferred_element_type=jnp.float32)

    out_ref[...] = xb + shared + aoe

    @pl.when(step == nsteps - 1)
    def _fin():
        lid = jax.lax.broadcasted_iota(jnp.int32, (E * _LANES, E), 0)
        eid = jax.lax.broadcasted_iota(jnp.int32, (E * _LANES, E), 1)
        selk = (lid // _LANES == eid).astype(jnp.float32)
        psum = jnp.dot(jnp.sum(pp_ref[...], axis=0, keepdims=True), selk,
                       preferred_element_type=jnp.float32)
        lsum = jnp.dot(jnp.sum(lp_ref[...], axis=0, keepdims=True), selk,
                       preferred_element_type=jnp.float32)
        n_f = jnp.float32(n_tokens)
        aux_ref[...] = (jnp.float32(E) / (n_f * n_f)
                        * jnp.sum(psum * lsum, keepdims=True))


def kernel(x, conv1_w, conv1_b, conv2_w, conv2_b, w_down, router_w, w_up):
    B, C, H, W = x.shape
    E, d_low, _ = w_up.shape
    hid = conv1_w.shape[0]
    N = B * H * W
    nsteps = N // _TN
    nw = _NC * _NS
    tpw = N // nw
    n_chunks = tpw // _LANES
    assert N % _TN == 0 and N % (nw * _LANES) == 0

    x_tok = x.transpose(0, 2, 3, 1).reshape(N, C)
    w1b = conv1_w.astype(jnp.bfloat16)
    w2b = conv2_w.astype(jnp.bfloat16)
    wdb = w_down.astype(jnp.bfloat16)
    eye = jnp.eye(E, dtype=x.dtype)
    rmat = jnp.kron(eye, router_w[0][:, None])
    selm = jnp.kron(eye, jnp.ones((1, d_low), x.dtype))
    wupf = w_up.reshape(E * d_low, C).astype(jnp.bfloat16)

    full = lambda r, c: pl.BlockSpec((r, c), lambda i: (0, 0))

    tn_lg = 1024
    logits = pl.pallas_call(
        _logits_body,
        grid=(N // tn_lg,),
        in_specs=[
            pl.BlockSpec((tn_lg, C), lambda i: (i, 0)),
            full(E * d_low, C), full(E * d_low, E),
        ],
        out_specs=pl.BlockSpec((E, tn_lg), lambda i: (0, i)),
        out_shape=jax.ShapeDtypeStruct((E, N), jnp.float32),
    )(x_tok, w_down, rmat)

    mesh = plsc.VectorSubcoreMesh(core_axis_name="c", subcore_axis_name="s")
    gate_flat, pp, lp = pl.kernel(
        functools.partial(_router_body, n_chunks, N, E),
        out_type=[
            jax.ShapeDtypeStruct((N * E,), jnp.float32),
            jax.ShapeDtypeStruct((nw * E * _LANES,), jnp.float32),
            jax.ShapeDtypeStruct((nw * E * _LANES,), jnp.float32),
        ],
        mesh=mesh,
        scratch_types=[
            pltpu.VMEM((tpw * E,), jnp.float32),
            pltpu.VMEM((tpw * E,), jnp.float32),
            pltpu.VMEM((E * _LANES,), jnp.float32),
            pltpu.VMEM((E * _LANES,), jnp.float32),
        ],
    )(logits.reshape(N * E))

    gate = gate_flat.reshape(E, N)
    pp2 = pp.reshape(nw, E * _LANES)
    lp2 = lp.reshape(nw, E * _LANES)

    tn_mix = 1536
    nsteps_mix = N // tn_mix
    out_tok, aux = pl.pallas_call(
        functools.partial(_mix_body, nsteps_mix, N),
        grid=(nsteps_mix,),
        in_specs=[
            pl.BlockSpec((tn_mix, C), lambda i: (i, 0)),
            full(hid, C), full(1, hid), full(C, hid), full(1, C),
            full(E * d_low, C), full(E, E * d_low), full(E * d_low, C),
            pl.BlockSpec((E, tn_mix), lambda i: (0, i)),
            full(nw, E * _LANES), full(nw, E * _LANES),
        ],
        out_specs=[
            pl.BlockSpec((tn_mix, C), lambda i: (i, 0)),
            full(1, 1),
        ],
        out_shape=[
            jax.ShapeDtypeStruct((N, C), jnp.float32),
            jax.ShapeDtypeStruct((1, 1), jnp.float32),
        ],
    )(x_tok, w1b, conv1_b[None, :], w2b, conv2_b[None, :], wdb, selm, wupf,
      gate, pp2, lp2)

    out = out_tok.reshape(B, H, W, C).transpose(0, 3, 1, 2)
    return (out, aux[0, 0])


# speedup vs baseline: 1.0480x; 1.0336x over previous
import functools

import jax
import jax.numpy as jnp
from jax import lax
from jax.experimental import pallas as pl
from jax.experimental.pallas import tpu as pltpu
from jax.experimental.pallas import tpu_sc as plsc

_TN = 512
_NC = 2
_NS = 16
_LANES = 16

_DN_NT = (((1,), (1,)), ((), ()))
_DN_TN = (((0,), (0,)), ((), ()))


def _gelu_exact(v):
    return 0.5 * v * (1.0 + jax.lax.erf(v * jnp.float32(0.7071067811865476)))


def _logits_body(x_ref, wd_ref, rmat_ref, logits_ref):
    rmat2 = lax.dot_general(wd_ref[...], rmat_ref[...], _DN_TN,
                            preferred_element_type=jnp.float32)
    logits_ref[...] = lax.dot_general(
        rmat2, x_ref[...], (((0,), (1,)), ((), ())),
        preferred_element_type=jnp.float32)


def _router_body(n_chunks, n_tokens, E, lg_hbm, gate_hbm, pp_hbm, lp_hbm,
                 lg_v, gt_v, pp_v, lp_v):
    wid = lax.axis_index("s") * _NC + lax.axis_index("c")
    tpw = n_chunks * _LANES
    base = wid * tpw
    for e in range(E):
        pltpu.sync_copy(lg_hbm.at[pl.ds(e * n_tokens + base, tpw)],
                        lg_v.at[pl.ds(e * tpw, tpw)])

    zf = jnp.zeros((_LANES,), jnp.float32)
    acc_p = [zf] * E
    acc_l = [zf] * E
    for j in range(n_chunks):
        off = j * _LANES
        ps = [lg_v[pl.ds(e * tpw + off, _LANES)] for e in range(E)]
        mx = ps[0]
        for e in range(1, E):
            mx = jnp.maximum(mx, ps[e])
        ps = [jnp.exp(v - mx) for v in ps]
        tot = ps[0]
        for e in range(1, E):
            tot = tot + ps[e]
        ps = [v / tot for v in ps]
        m1 = ps[0]
        for e in range(1, E):
            m1 = jnp.maximum(m1, ps[e])
        i1 = jnp.zeros((_LANES,), jnp.int32)
        for e in range(E - 1, -1, -1):
            i1 = jnp.where(ps[e] == m1, e, i1)
        rest = [jnp.where(i1 == e, -1.0, ps[e]) for e in range(E)]
        m2 = rest[0]
        for e in range(1, E):
            m2 = jnp.maximum(m2, rest[e])
        i2 = jnp.zeros((_LANES,), jnp.int32)
        for e in range(E - 1, -1, -1):
            i2 = jnp.where(rest[e] == m2, e, i2)
        denom = m1 + m2
        for e in range(E):
            sel = (i1 == e) | (i2 == e)
            gt_v[pl.ds(e * tpw + off, _LANES)] = (
                jnp.where(sel, ps[e], 0.0) / denom)
            acc_p[e] = acc_p[e] + ps[e]
            acc_l[e] = acc_l[e] + jnp.where(sel, 1.0, 0.0)

    for e in range(E):
        pp_v[pl.ds(e * _LANES, _LANES)] = acc_p[e]
        lp_v[pl.ds(e * _LANES, _LANES)] = acc_l[e]
    for e in range(E):
        pltpu.sync_copy(gt_v.at[pl.ds(e * tpw, tpw)],
                        gate_hbm.at[pl.ds(e * n_tokens + base, tpw)])
    npad = E * _LANES
    pltpu.sync_copy(pp_v, pp_hbm.at[pl.ds(wid * npad, npad)])
    pltpu.sync_copy(lp_v, lp_hbm.at[pl.ds(wid * npad, npad)])


def _mix_body(nsteps, n_tokens, x_ref, w1_ref, b1_ref, w2_ref, b2_ref,
              wd_ref, sel_ref, wup_ref, gate_ref, pp_ref, lp_ref,
              out_ref, aux_ref):
    step = pl.program_id(0)
    xb = x_ref[...]
    E = sel_ref.shape[0]

    xb_h = xb.astype(jnp.bfloat16)
    h = _gelu_exact(
        lax.dot_general(xb_h, w1_ref[...], _DN_NT,
                        preferred_element_type=jnp.float32)
        + b1_ref[...])
    shared = (lax.dot_general(h.astype(jnp.bfloat16), w2_ref[...], _DN_NT,
                              preferred_element_type=jnp.float32)
              + b2_ref[...])

    feats = lax.dot_general(xb_h, wd_ref[...], _DN_NT,
                            preferred_element_type=jnp.float32)
    gate_big = lax.dot_general(gate_ref[...], sel_ref[...], _DN_TN,
                               preferred_element_type=jnp.float32)
    wf = _gelu_exact(feats) * gate_big
    aoe = jnp.dot(wf.astype(jnp.bfloat16), wup_ref[...],
                  preferred_element_type=jnp.float32)

    out_ref[...] = xb + shared + aoe

    @pl.when(step == nsteps - 1)
    def _fin():
        lid = jax.lax.broadcasted_iota(jnp.int32, (E * _LANES, E), 0)
        eid = jax.lax.broadcasted_iota(jnp.int32, (E * _LANES, E), 1)
        selk = (lid // _LANES == eid).astype(jnp.float32)
        psum = jnp.dot(jnp.sum(pp_ref[...], axis=0, keepdims=True), selk,
                       preferred_element_type=jnp.float32)
        lsum = jnp.dot(jnp.sum(lp_ref[...], axis=0, keepdims=True), selk,
                       preferred_element_type=jnp.float32)
        n_f = jnp.float32(n_tokens)
        aux_ref[...] = (jnp.float32(E) / (n_f * n_f)
                        * jnp.sum(psum * lsum, keepdims=True))


def kernel(x, conv1_w, conv1_b, conv2_w, conv2_b, w_down, router_w, w_up):
    B, C, H, W = x.shape
    E, d_low, _ = w_up.shape
    hid = conv1_w.shape[0]
    N = B * H * W
    nsteps = N // _TN
    nw = _NC * _NS
    tpw = N // nw
    n_chunks = tpw // _LANES
    assert N % _TN == 0 and N % (nw * _LANES) == 0

    x_tok = x.transpose(0, 2, 3, 1).reshape(N, C)
    w1b = conv1_w.astype(jnp.bfloat16)
    w2b = conv2_w.astype(jnp.bfloat16)
    wdb = w_down.astype(jnp.bfloat16)
    eye = jnp.eye(E, dtype=x.dtype)
    rmat = jnp.kron(eye, router_w[0][:, None])
    selm = jnp.kron(eye, jnp.ones((1, d_low), x.dtype))
    wupf = w_up.reshape(E * d_low, C).astype(jnp.bfloat16)

    full = lambda r, c: pl.BlockSpec((r, c), lambda i: (0, 0))

    tn_lg = 2304
    logits = pl.pallas_call(
        _logits_body,
        grid=(N // tn_lg,),
        in_specs=[
            pl.BlockSpec((tn_lg, C), lambda i: (i, 0)),
            full(E * d_low, C), full(E * d_low, E),
        ],
        out_specs=pl.BlockSpec((E, tn_lg), lambda i: (0, i)),
        out_shape=jax.ShapeDtypeStruct((E, N), jnp.float32),
    )(x_tok, w_down, rmat)

    mesh = plsc.VectorSubcoreMesh(core_axis_name="c", subcore_axis_name="s")
    gate_flat, pp, lp = pl.kernel(
        functools.partial(_router_body, n_chunks, N, E),
        out_type=[
            jax.ShapeDtypeStruct((N * E,), jnp.float32),
            jax.ShapeDtypeStruct((nw * E * _LANES,), jnp.float32),
            jax.ShapeDtypeStruct((nw * E * _LANES,), jnp.float32),
        ],
        mesh=mesh,
        scratch_types=[
            pltpu.VMEM((tpw * E,), jnp.float32),
            pltpu.VMEM((tpw * E,), jnp.float32),
            pltpu.VMEM((E * _LANES,), jnp.float32),
            pltpu.VMEM((E * _LANES,), jnp.float32),
        ],
    )(logits.reshape(N * E))

    gate = gate_flat.reshape(E, N)
    pp2 = pp.reshape(nw, E * _LANES)
    lp2 = lp.reshape(nw, E * _LANES)

    tn_mix = 1536
    nsteps_mix = N // tn_mix
    out_tok, aux = pl.pallas_call(
        functools.partial(_mix_body, nsteps_mix, N),
        grid=(nsteps_mix,),
        in_specs=[
            pl.BlockSpec((tn_mix, C), lambda i: (i, 0)),
            full(hid, C), full(1, hid), full(C, hid), full(1, C),
            full(E * d_low, C), full(E, E * d_low), full(E * d_low, C),
            pl.BlockSpec((E, tn_mix), lambda i: (0, i)),
            full(nw, E * _LANES), full(nw, E * _LANES),
        ],
        out_specs=[
            pl.BlockSpec((tn_mix, C), lambda i: (i, 0)),
            full(1, 1),
        ],
        out_shape=[
            jax.ShapeDtypeStruct((N, C), jnp.float32),
            jax.ShapeDtypeStruct((1, 1), jnp.float32),
        ],
    )(x_tok, w1b, conv1_b[None, :], w2b, conv2_b[None, :], wdb, selm, wupf,
      gate, pp2, lp2)

    out = out_tok.reshape(B, H, W, C).transpose(0, 3, 1, 2)
    return (out, aux[0, 0])


# speedup vs baseline: 1.0489x; 1.0009x over previous
import functools

import jax
import jax.numpy as jnp
from jax import lax
from jax.experimental import pallas as pl
from jax.experimental.pallas import tpu as pltpu
from jax.experimental.pallas import tpu_sc as plsc

_NC = 2
_NS = 16
_LANES = 16

_DN_NT = (((1,), (1,)), ((), ()))
_DN_TN = (((0,), (0,)), ((), ()))


def _gelu_exact(v):
    return 0.5 * v * (1.0 + jax.lax.erf(v * jnp.float32(0.7071067811865476)))


def _logits_body(x_ref, wd_ref, rmat_ref, logits_ref):
    rmat2 = lax.dot_general(wd_ref[...], rmat_ref[...], _DN_TN,
                            preferred_element_type=jnp.float32)
    logits_ref[...] = lax.dot_general(
        rmat2, x_ref[...], (((0,), (1,)), ((), ())),
        preferred_element_type=jnp.float32)


def _router_body(n_chunks, n_tokens, E, lg_hbm, gate_hbm, pp_hbm, lp_hbm,
                 lg_v, gt_v, pp_v, lp_v):
    wid = lax.axis_index("s") * _NC + lax.axis_index("c")
    tpw = n_chunks * _LANES
    base = wid * tpw
    for e in range(E):
        pltpu.sync_copy(lg_hbm.at[pl.ds(e * n_tokens + base, tpw)],
                        lg_v.at[pl.ds(e * tpw, tpw)])

    zf = jnp.zeros((_LANES,), jnp.float32)
    acc_p = [zf] * E
    acc_l = [zf] * E
    for j in range(n_chunks):
        off = j * _LANES
        ps = [lg_v[pl.ds(e * tpw + off, _LANES)] for e in range(E)]
        mx = ps[0]
        for e in range(1, E):
            mx = jnp.maximum(mx, ps[e])
        ps = [jnp.exp(v - mx) for v in ps]
        tot = ps[0]
        for e in range(1, E):
            tot = tot + ps[e]
        ps = [v / tot for v in ps]
        m1 = ps[0]
        for e in range(1, E):
            m1 = jnp.maximum(m1, ps[e])
        i1 = jnp.zeros((_LANES,), jnp.int32)
        for e in range(E - 1, -1, -1):
            i1 = jnp.where(ps[e] == m1, e, i1)
        rest = [jnp.where(i1 == e, -1.0, ps[e]) for e in range(E)]
        m2 = rest[0]
        for e in range(1, E):
            m2 = jnp.maximum(m2, rest[e])
        i2 = jnp.zeros((_LANES,), jnp.int32)
        for e in range(E - 1, -1, -1):
            i2 = jnp.where(rest[e] == m2, e, i2)
        denom = m1 + m2
        for e in range(E):
            sel = (i1 == e) | (i2 == e)
            gt_v[pl.ds(e * tpw + off, _LANES)] = (
                jnp.where(sel, ps[e], 0.0) / denom)
            acc_p[e] = acc_p[e] + ps[e]
            acc_l[e] = acc_l[e] + jnp.where(sel, 1.0, 0.0)

    for e in range(E):
        pp_v[pl.ds(e * _LANES, _LANES)] = acc_p[e]
        lp_v[pl.ds(e * _LANES, _LANES)] = acc_l[e]
    for e in range(E):
        pltpu.sync_copy(gt_v.at[pl.ds(e * tpw, tpw)],
                        gate_hbm.at[pl.ds(e * n_tokens + base, tpw)])
    npad = E * _LANES
    pltpu.sync_copy(pp_v, pp_hbm.at[pl.ds(wid * npad, npad)])
    pltpu.sync_copy(lp_v, lp_hbm.at[pl.ds(wid * npad, npad)])


def _mix_body(nsteps, n_tokens, x_ref, w1_ref, b1_ref, w2_ref, b2_ref,
              wd_ref, sel_ref, wup_ref, gate_ref, pp_ref, lp_ref,
              out_ref, aux_ref):
    step = pl.program_id(0)
    xb = x_ref[...]
    E = sel_ref.shape[0]

    xb_h = xb.astype(jnp.bfloat16)
    h = _gelu_exact(
        lax.dot_general(xb_h, w1_ref[...], _DN_NT,
                        preferred_element_type=jnp.float32)
        + b1_ref[...])
    shared = (lax.dot_general(h.astype(jnp.bfloat16), w2_ref[...], _DN_NT,
                              preferred_element_type=jnp.float32)
              + b2_ref[...])

    feats = lax.dot_general(xb_h, wd_ref[...], _DN_NT,
                            preferred_element_type=jnp.float32)
    gate_big = lax.dot_general(gate_ref[...], sel_ref[...], _DN_TN,
                               preferred_element_type=jnp.float32)
    wf = _gelu_exact(feats) * gate_big
    aoe = jnp.dot(wf.astype(jnp.bfloat16), wup_ref[...],
                  preferred_element_type=jnp.float32)

    out_ref[...] = xb + shared + aoe

    @pl.when(step == nsteps - 1)
    def _fin():
        lid = jax.lax.broadcasted_iota(jnp.int32, (E * _LANES, E), 0)
        eid = jax.lax.broadcasted_iota(jnp.int32, (E * _LANES, E), 1)
        selk = (lid // _LANES == eid).astype(jnp.float32)
        psum = jnp.dot(jnp.sum(pp_ref[...], axis=0, keepdims=True), selk,
                       preferred_element_type=jnp.float32)
        lsum = jnp.dot(jnp.sum(lp_ref[...], axis=0, keepdims=True), selk,
                       preferred_element_type=jnp.float32)
        n_f = jnp.float32(n_tokens)
        aux_ref[...] = (jnp.float32(E) / (n_f * n_f)
                        * jnp.sum(psum * lsum, keepdims=True))


def kernel(x, conv1_w, conv1_b, conv2_w, conv2_b, w_down, router_w, w_up):
    B, C, H, W = x.shape
    E, d_low, _ = w_up.shape
    hid = conv1_w.shape[0]
    N = B * H * W
    nw = _NC * _NS
    tpw = N // nw
    n_chunks = tpw // _LANES
    assert N % (nw * _LANES) == 0

    x_tok = x.transpose(0, 2, 3, 1).reshape(N, C)
    w1b = conv1_w.astype(jnp.bfloat16)
    w2b = conv2_w.astype(jnp.bfloat16)
    wdb = w_down.astype(jnp.bfloat16)
    eye = jnp.eye(E, dtype=x.dtype)
    rmat = jnp.kron(eye, router_w[0][:, None])
    selm = jnp.kron(eye, jnp.ones((1, d_low), x.dtype))
    wupf = w_up.reshape(E * d_low, C).astype(jnp.bfloat16)

    full = lambda r, c: pl.BlockSpec((r, c), lambda i: (0, 0))

    tn_lg = 2304
    logits = pl.pallas_call(
        _logits_body,
        grid=(N // tn_lg,),
        in_specs=[
            pl.BlockSpec((tn_lg, C), lambda i: (i, 0)),
            full(E * d_low, C), full(E * d_low, E),
        ],
        out_specs=pl.BlockSpec((E, tn_lg), lambda i: (0, i)),
        out_shape=jax.ShapeDtypeStruct((E, N), jnp.float32),
    )(x_tok, w_down, rmat)

    mesh = plsc.VectorSubcoreMesh(core_axis_name="c", subcore_axis_name="s")
    gate_flat, pp, lp = pl.kernel(
        functools.partial(_router_body, n_chunks, N, E),
        out_type=[
            jax.ShapeDtypeStruct((N * E,), jnp.float32),
            jax.ShapeDtypeStruct((nw * E * _LANES,), jnp.float32),
            jax.ShapeDtypeStruct((nw * E * _LANES,), jnp.float32),
        ],
        mesh=mesh,
        scratch_types=[
            pltpu.VMEM((tpw * E,), jnp.float32),
            pltpu.VMEM((tpw * E,), jnp.float32),
            pltpu.VMEM((E * _LANES,), jnp.float32),
            pltpu.VMEM((E * _LANES,), jnp.float32),
        ],
    )(logits.reshape(N * E))

    gate = gate_flat.reshape(E, N)
    pp2 = pp.reshape(nw, E * _LANES)
    lp2 = lp.reshape(nw, E * _LANES)

    tn_mix = 1536
    nsteps_mix = N // tn_mix
    out_tok, aux = pl.pallas_call(
        functools.partial(_mix_body, nsteps_mix, N),
        grid=(nsteps_mix,),
        in_specs=[
            pl.BlockSpec((tn_mix, C), lambda i: (i, 0)),
            full(hid, C), full(1, hid), full(C, hid), full(1, C),
            full(E * d_low, C), full(E, E * d_low), full(E * d_low, C),
            pl.BlockSpec((E, tn_mix), lambda i: (0, i)),
            full(nw, E * _LANES), full(nw, E * _LANES),
        ],
        out_specs=[
            pl.BlockSpec((tn_mix, C), lambda i: (i, 0)),
            full(1, 1),
        ],
        out_shape=[
            jax.ShapeDtypeStruct((N, C), jnp.float32),
            jax.ShapeDtypeStruct((1, 1), jnp.float32),
        ],
    )(x_tok, w1b, conv1_b[None, :], w2b, conv2_b[None, :], wdb, selm, wupf,
      gate, pp2, lp2)

    out = out_tok.reshape(B, H, W, C).transpose(0, 3, 1, 2)
    return (out, aux[0, 0])
